# Initial kernel scaffold; baseline (speedup 1.0000x reference)
import functools

import jax
import jax.numpy as jnp
from jax import lax
from jax.experimental import pallas as pl
from jax.experimental.pallas import tpu as pltpu
from jax.experimental.pallas import tpu_sc as plsc

N = 10000
E = 320000
C_IN = 128
C_V = 64
H = 8

NC = 2
NS = 16
B = 80
EDGES_PER_TILE = E // (NC * NS)
BLOCKS_PER_TILE = EDGES_PER_TILE // B
ACC_D = 80
ROWS_PER_TILE = N // NS
TC_BLK = 1000


def _pre_body(x_ref, wkv_ref, wq_ref, vk_ref, q_ref):
    x = x_ref[...]
    vk_ref[...] = jnp.dot(x, wkv_ref[...], preferred_element_type=jnp.float32)
    q_ref[...] = jnp.dot(x, wq_ref[...], preferred_element_type=jnp.float32)


def _post_body(x_ref, a0_ref, a1_ref, wpt_ref, wpb_ref, r_ref, out_ref):
    a = a0_ref[...] + a1_ref[...]
    w = a[:, :C_V]
    den = a[:, C_V:C_V + H]
    den_rep = jnp.dot(den, r_ref[...], preferred_element_type=jnp.float32)
    feat = w / (den_rep + 1e-9)
    out_ref[...] = (
        jnp.dot(x_ref[...], wpt_ref[...], preferred_element_type=jnp.float32)
        + jnp.dot(feat, wpb_ref[...], preferred_element_type=jnp.float32))


def _sc_body(vk_hbm, q_hbm, src_hbm, dst_hbm, out_hbm,
             src_idx, dst_idx, vkbuf, qbuf, wbuf, zbuf, acc_sh, sem1, sem2):
    cid = lax.axis_index("c")
    sid = lax.axis_index("s")
    zeros16 = jnp.zeros((16,), jnp.float32)

    def zero_z(i, _):
        for cpart in range(ACC_D // 16):
            zbuf[i, pl.ds(cpart * 16, 16)] = zeros16
        return 0
    lax.fori_loop(0, ROWS_PER_TILE // 5, zero_z, 0)
    for r in range(5):
        pltpu.sync_copy(
            zbuf, acc_sh.at[pl.ds(sid * ROWS_PER_TILE + r * (ROWS_PER_TILE // 5),
                                  ROWS_PER_TILE // 5)])

    def zero_w(i, _):
        wbuf[i, pl.ds(C_V, 16)] = zeros16
        return 0
    lax.fori_loop(0, B, zero_w, 0)

    plsc.subcore_barrier()

    tile_edge_base = cid * (E // NC) + sid * EDGES_PER_TILE

    def block_body(blk, _):
        base = tile_edge_base + blk * B
        pltpu.sync_copy(src_hbm.at[pl.ds(base, B)], src_idx)
        pltpu.sync_copy(dst_hbm.at[pl.ds(base, B)], dst_idx)
        cp1 = pltpu.async_copy(vk_hbm.at[src_idx], vkbuf, sem1)
        cp2 = pltpu.async_copy(q_hbm.at[dst_idx], qbuf, sem2)
        cp1.wait()
        cp2.wait()

        def group_body(g, _):
            eoff = lax.iota(jnp.int32, 16) + g * 16
            dots = [zeros16] * H
            for c in range(C_V):
                col_k = jnp.full((16,), C_V + c, jnp.int32)
                col_q = jnp.full((16,), c, jnp.int32)
                kc = plsc.load_gather(vkbuf, [eoff, col_k])
                qc = plsc.load_gather(qbuf, [eoff, col_q])
                dots[c // 8] = dots[c // 8] + kc * qc
            ps = []
            for h in range(H):
                p = jnp.exp(dots[h] * 0.125)
                ps.append(p)
                plsc.store_scatter(wbuf, [eoff, jnp.full((16,), C_V + h, jnp.int32)], p)
            for c in range(C_V):
                col = jnp.full((16,), c, jnp.int32)
                vc = plsc.load_gather(vkbuf, [eoff, col])
                plsc.store_scatter(wbuf, [eoff, col], vc * ps[c // 8])
            return 0

        lax.fori_loop(0, B // 16, group_body, 0)
        pltpu.sync_copy(wbuf, acc_sh.at[dst_idx], add=True)
        return 0

    lax.fori_loop(0, BLOCKS_PER_TILE, block_body, 0)

    plsc.subcore_barrier()
    row0 = sid * ROWS_PER_TILE
    pltpu.sync_copy(acc_sh.at[pl.ds(row0, ROWS_PER_TILE)],
                    out_hbm.at[pl.ds(cid * N + row0, ROWS_PER_TILE)])


def kernel(node_feats_0, edge_index, W_kv, W_q, W_proj):
    x = node_feats_0[:, :, 0]

    vk, q = pl.pallas_call(
        _pre_body,
        grid=(N // TC_BLK,),
        in_specs=[
            pl.BlockSpec((TC_BLK, C_IN), lambda i: (i, 0)),
            pl.BlockSpec((C_IN, 2 * C_V), lambda i: (0, 0)),
            pl.BlockSpec((C_IN, C_V), lambda i: (0, 0)),
        ],
        out_specs=[
            pl.BlockSpec((TC_BLK, 2 * C_V), lambda i: (i, 0)),
            pl.BlockSpec((TC_BLK, C_V), lambda i: (i, 0)),
        ],
        out_shape=[
            jax.ShapeDtypeStruct((N, 2 * C_V), jnp.float32),
            jax.ShapeDtypeStruct((N, C_V), jnp.float32),
        ],
    )(x, W_kv, W_q)

    src = edge_index[0]
    dst = edge_index[1]

    sc_edges = pl.kernel(
        _sc_body,
        out_type=jax.ShapeDtypeStruct((NC * N, ACC_D), jnp.float32),
        mesh=plsc.VectorSubcoreMesh(core_axis_name="c", subcore_axis_name="s"),
        scratch_types=[
            pltpu.VMEM((B,), jnp.int32),
            pltpu.VMEM((B,), jnp.int32),
            pltpu.VMEM((B, 2 * C_V), jnp.float32),
            pltpu.VMEM((B, C_V), jnp.float32),
            pltpu.VMEM((B, ACC_D), jnp.float32),
            pltpu.VMEM((ROWS_PER_TILE // 5, ACC_D), jnp.float32),
            pltpu.VMEM_SHARED((N, ACC_D), jnp.float32),
            pltpu.SemaphoreType.DMA,
            pltpu.SemaphoreType.DMA,
        ],
    )
    acc = sc_edges(vk, q, src, dst)

    wp_top = W_proj[:C_IN]
    wp_bot = W_proj[C_IN:]
    r_mat = jnp.kron(jnp.eye(H, dtype=jnp.float32),
                     jnp.ones((1, H), dtype=jnp.float32))

    out2d = pl.pallas_call(
        _post_body,
        grid=(N // TC_BLK,),
        in_specs=[
            pl.BlockSpec((TC_BLK, C_IN), lambda i: (i, 0)),
            pl.BlockSpec((TC_BLK, ACC_D), lambda i: (i, 0)),
            pl.BlockSpec((TC_BLK, ACC_D), lambda i: (i, 0)),
            pl.BlockSpec((C_IN, C_IN), lambda i: (0, 0)),
            pl.BlockSpec((C_V, C_IN), lambda i: (0, 0)),
            pl.BlockSpec((H, C_V), lambda i: (0, 0)),
        ],
        out_specs=pl.BlockSpec((TC_BLK, C_IN), lambda i: (i, 0)),
        out_shape=jax.ShapeDtypeStruct((N, C_IN), jnp.float32),
    )(x, acc[:N], acc[N:], wp_top, wp_bot, r_mat)

    return out2d[:, :, None]


# speedup vs baseline: 24.4278x; 24.4278x over previous
import functools

import jax
import jax.numpy as jnp
from jax import lax
from jax.experimental import pallas as pl
from jax.experimental.pallas import tpu as pltpu
from jax.experimental.pallas import tpu_sc as plsc

N = 10000
E = 320000
C_IN = 128
C_V = 64
H = 8

NC = 2
NS = 16
B = 80
EDGES_PER_TILE = E // (NC * NS)
BLOCKS_PER_TILE = EDGES_PER_TILE // B
ACC_D = 128
N_PAD = 10240
ROWS_PER_TILE = N_PAD // NS
TC_BLK = 1000


def _pre_body(x_ref, wkv_ref, wq_ref, vk_ref, q_ref):
    x = x_ref[...]
    vk_ref[...] = jnp.dot(x, wkv_ref[...], preferred_element_type=jnp.float32)
    q_ref[...] = jnp.dot(x, wq_ref[...], preferred_element_type=jnp.float32)


def _post_body(x_ref, a0_ref, a1_ref, wpt_ref, wpb_ref, r_ref, out_ref):
    a = a0_ref[...] + a1_ref[...]
    w = a[:, :C_V]
    den = a[:, C_V:C_V + H]
    den_rep = jnp.dot(den, r_ref[...], preferred_element_type=jnp.float32)
    feat = w / (den_rep + 1e-9)
    out_ref[...] = (
        jnp.dot(x_ref[...], wpt_ref[...], preferred_element_type=jnp.float32)
        + jnp.dot(feat, wpb_ref[...], preferred_element_type=jnp.float32))


def _sc_body(vk_hbm, q_hbm, src_hbm, dst_hbm, out_hbm,
             src_idx, dst_idx, vkbuf, qbuf, wbuf, zbuf, acc_sh, sem1, sem2):
    cid = lax.axis_index("c")
    sid = lax.axis_index("s")
    zeros16 = jnp.zeros((16,), jnp.float32)

    def zero_z(i, _):
        for cpart in range(ACC_D // 16):
            zbuf[i, pl.ds(cpart * 16, 16)] = zeros16
        return 0
    lax.fori_loop(0, ROWS_PER_TILE // 5, zero_z, 0)
    for r in range(5):
        pltpu.sync_copy(
            zbuf, acc_sh.at[pl.ds(sid * ROWS_PER_TILE + r * (ROWS_PER_TILE // 5),
                                  ROWS_PER_TILE // 5)])

    def zero_w(i, _):
        for cpart in range((ACC_D - C_V) // 16):
            wbuf[i, pl.ds(C_V + cpart * 16, 16)] = zeros16
        return 0
    lax.fori_loop(0, B, zero_w, 0)

    plsc.subcore_barrier()

    tile_edge_base = cid * (E // NC) + sid * EDGES_PER_TILE

    def block_body(blk, _):
        base = tile_edge_base + blk * B
        pltpu.sync_copy(src_hbm.at[pl.ds(base, B)], src_idx)
        pltpu.sync_copy(dst_hbm.at[pl.ds(base, B)], dst_idx)
        cp1 = pltpu.async_copy(vk_hbm.at[src_idx], vkbuf, sem1)
        cp2 = pltpu.async_copy(q_hbm.at[dst_idx], qbuf, sem2)
        cp1.wait()
        cp2.wait()

        def group_body(g, _):
            eoff = lax.iota(jnp.int32, 16) + g * 16
            dots = [zeros16] * H
            for c in range(C_V):
                col_k = jnp.full((16,), C_V + c, jnp.int32)
                col_q = jnp.full((16,), c, jnp.int32)
                kc = plsc.load_gather(vkbuf, [eoff, col_k])
                qc = plsc.load_gather(qbuf, [eoff, col_q])
                dots[c // 8] = dots[c // 8] + kc * qc
            ps = []
            for h in range(H):
                p = jnp.exp(dots[h] * 0.125)
                ps.append(p)
                plsc.store_scatter(wbuf, [eoff, jnp.full((16,), C_V + h, jnp.int32)], p)
            for c in range(C_V):
                col = jnp.full((16,), c, jnp.int32)
                vc = plsc.load_gather(vkbuf, [eoff, col])
                plsc.store_scatter(wbuf, [eoff, col], vc * ps[c // 8])
            return 0

        lax.fori_loop(0, B // 16, group_body, 0)
        pltpu.sync_copy(wbuf, acc_sh.at[dst_idx], add=True)
        return 0

    lax.fori_loop(0, BLOCKS_PER_TILE, block_body, 0)

    plsc.subcore_barrier()
    row0 = sid * ROWS_PER_TILE
    pltpu.sync_copy(acc_sh.at[pl.ds(row0, ROWS_PER_TILE)],
                    out_hbm.at[pl.ds(cid * N_PAD + row0, ROWS_PER_TILE)])


def kernel(node_feats_0, edge_index, W_kv, W_q, W_proj):
    x = node_feats_0[:, :, 0]

    vk, q = pl.pallas_call(
        _pre_body,
        grid=(N // TC_BLK,),
        in_specs=[
            pl.BlockSpec((TC_BLK, C_IN), lambda i: (i, 0)),
            pl.BlockSpec((C_IN, 2 * C_V), lambda i: (0, 0)),
            pl.BlockSpec((C_IN, 2 * C_V), lambda i: (0, 0)),
        ],
        out_specs=[
            pl.BlockSpec((TC_BLK, 2 * C_V), lambda i: (i, 0)),
            pl.BlockSpec((TC_BLK, 2 * C_V), lambda i: (i, 0)),
        ],
        out_shape=[
            jax.ShapeDtypeStruct((N, 2 * C_V), jnp.float32),
            jax.ShapeDtypeStruct((N, 2 * C_V), jnp.float32),
        ],
    )(x, W_kv, jnp.pad(W_q, ((0, 0), (0, C_V))))

    src = edge_index[0]
    dst = edge_index[1]

    sc_edges = pl.kernel(
        _sc_body,
        out_type=jax.ShapeDtypeStruct((NC * N_PAD, ACC_D), jnp.float32),
        mesh=plsc.VectorSubcoreMesh(core_axis_name="c", subcore_axis_name="s"),
        compiler_params=pltpu.CompilerParams(needs_layout_passes=False),
        scratch_types=[
            pltpu.VMEM((B,), jnp.int32),
            pltpu.VMEM((B,), jnp.int32),
            pltpu.VMEM((B, 2 * C_V), jnp.float32),
            pltpu.VMEM((B, 2 * C_V), jnp.float32),
            pltpu.VMEM((B, ACC_D), jnp.float32),
            pltpu.VMEM((ROWS_PER_TILE // 5, ACC_D), jnp.float32),
            pltpu.VMEM_SHARED((N_PAD, ACC_D), jnp.float32),
            pltpu.SemaphoreType.DMA,
            pltpu.SemaphoreType.DMA,
        ],
    )
    acc = sc_edges(vk, q, src, dst)

    wp_top = W_proj[:C_IN]
    wp_bot = W_proj[C_IN:]
    r_mat = jnp.kron(jnp.eye(H, dtype=jnp.float32),
                     jnp.ones((1, H), dtype=jnp.float32))

    out2d = pl.pallas_call(
        _post_body,
        grid=(N // TC_BLK,),
        in_specs=[
            pl.BlockSpec((TC_BLK, C_IN), lambda i: (i, 0)),
            pl.BlockSpec((TC_BLK, ACC_D), lambda i: (i, 0)),
            pl.BlockSpec((TC_BLK, ACC_D), lambda i: (i, 0)),
            pl.BlockSpec((C_IN, C_IN), lambda i: (0, 0)),
            pl.BlockSpec((C_V, C_IN), lambda i: (0, 0)),
            pl.BlockSpec((H, C_V), lambda i: (0, 0)),
        ],
        out_specs=pl.BlockSpec((TC_BLK, C_IN), lambda i: (i, 0)),
        out_shape=jax.ShapeDtypeStruct((N, C_IN), jnp.float32),
    )(x, acc[:N], acc[N_PAD:N_PAD + N], wp_top, wp_bot, r_mat)

    return out2d[:, :, None]
